# baseline (device time: 150551 ns/iter reference)
import functools
import sys

import jax
import jax.numpy as jnp
from jax import lax
from jax.experimental import pallas as pl
from jax.experimental.pallas import tpu as pltpu

N_DEV = 16

try:
    for _d in jax.devices():
        print(
            "DEV",
            _d.id,
            getattr(_d, "coords", None),
            getattr(_d, "core_on_chip", None),
            file=sys.stderr,
        )
except Exception:
    pass


def kernel(A, B):
    M, _ = A.shape
    _, N = B.shape
    CH = M // N_DEV

    def body(a_ref, b_ref, out_ref, p_ref, rs_buf, send_sems, rs_sems, ag_sems):
        my = lax.axis_index("i")
        left = lax.rem(my + N_DEV - 1, N_DEV)
        right = lax.rem(my + 1, N_DEV)

        barrier_sem = pltpu.get_barrier_semaphore()
        for nbr in (left, right):
            pl.semaphore_signal(
                barrier_sem, inc=1,
                device_id=(nbr,), device_id_type=pl.DeviceIdType.MESH,
            )
        pl.semaphore_wait(barrier_sem, 2)

        p_ref[...] = jnp.dot(
            a_ref[...].astype(jnp.bfloat16),
            b_ref[...].astype(jnp.bfloat16),
            preferred_element_type=jnp.float32,
        )

        for h in range(N_DEV - 1):
            s_chunk = lax.rem(my - h + N_DEV, N_DEV)
            r_chunk = lax.rem(my - h - 1 + N_DEV, N_DEV)
            rdma = pltpu.make_async_remote_copy(
                src_ref=p_ref.at[pl.ds(s_chunk * CH, CH), :],
                dst_ref=rs_buf.at[h],
                send_sem=send_sems.at[h],
                recv_sem=rs_sems.at[h],
                device_id=(right,),
                device_id_type=pl.DeviceIdType.MESH,
            )
            rdma.start()
            rdma.wait()
            rows = pl.ds(r_chunk * CH, CH)
            p_ref[rows, :] += rs_buf[h]

        own_rows = pl.ds(lax.rem(my + 1, N_DEV) * CH, CH)
        out_ref[own_rows, :] = p_ref[own_rows, :]

        for h in range(N_DEV - 1):
            g_chunk = lax.rem(my + 1 - h + N_DEV, N_DEV)
            rows = pl.ds(g_chunk * CH, CH)
            rdma = pltpu.make_async_remote_copy(
                src_ref=out_ref.at[rows, :],
                dst_ref=out_ref.at[rows, :],
                send_sem=send_sems.at[h],
                recv_sem=ag_sems.at[h],
                device_id=(right,),
                device_id_type=pl.DeviceIdType.MESH,
            )
            rdma.start()
            rdma.wait()

        @functools.partial(pl.run_scoped, exit_sem=pltpu.SemaphoreType.REGULAR)
        def _(exit_sem):
            for nbr in (left, right):
                pl.semaphore_signal(
                    exit_sem, inc=1,
                    device_id=(nbr,), device_id_type=pl.DeviceIdType.MESH,
                )
            pl.semaphore_wait(exit_sem, 2)

    return pl.pallas_call(
        body,
        out_shape=jax.ShapeDtypeStruct((M, N), jnp.float32),
        in_specs=[
            pl.BlockSpec(memory_space=pltpu.VMEM),
            pl.BlockSpec(memory_space=pltpu.VMEM),
        ],
        out_specs=pl.BlockSpec(memory_space=pltpu.VMEM),
        scratch_shapes=[
            pltpu.VMEM((M, N), jnp.float32),
            pltpu.VMEM((N_DEV - 1, CH, N), jnp.float32),
            pltpu.SemaphoreType.DMA((N_DEV - 1,)),
            pltpu.SemaphoreType.DMA((N_DEV - 1,)),
            pltpu.SemaphoreType.DMA((N_DEV - 1,)),
        ],
        compiler_params=pltpu.CompilerParams(collective_id=0),
    )(A, B)


# device time: 57686 ns/iter; 2.6098x vs baseline; 2.6098x over previous
import functools

import jax
import jax.numpy as jnp
from jax import lax
from jax.experimental import pallas as pl
from jax.experimental.pallas import tpu as pltpu

N_DEV = 16
NP = 4
MC = 256
HC = 512


def kernel(A, B):
    M, _ = A.shape
    _, N = B.shape

    f32 = jnp.float32
    bf16 = jnp.bfloat16

    def body(
        a_ref, b_ref, out_ref,
        p_ref, r_ref, sb_cw, sb_ccw, rb_cw, rb_ccw,
        zsa, zra, zsb, zrb,
        prs_send_cw, prs_recv_cw, prs_send_ccw, prs_recv_ccw,
        z_send, z_recv,
        pag_send_cw, pag_recv_cw, pag_send_ccw, pag_recv_ccw,
    ):
        my = lax.axis_index("i")
        qi = lax.rem(my, NP)
        zi = lax.div(my, NP)
        b0 = lax.rem(zi, 2)
        b1 = lax.div(zi, 2)

        right = zi * NP + lax.rem(qi + 1, NP)
        left = zi * NP + lax.rem(qi + 3, NP)
        dz1 = (zi + 1 - 2 * b0) * NP + qi
        dz2 = (zi + 2 - 4 * b1) * NP + qi

        def xsig(sem, dev):
            pl.semaphore_signal(
                sem, inc=1, device_id=(dev,),
                device_id_type=pl.DeviceIdType.MESH,
            )

        barrier_sem = pltpu.get_barrier_semaphore()
        for nbr in (left, right, dz1, dz2):
            xsig(barrier_sem, nbr)
        pl.semaphore_wait(barrier_sem, 4)

        p_ref[...] = jnp.dot(
            a_ref[...].astype(bf16),
            b_ref[...].astype(bf16),
            preferred_element_type=f32,
        )

        def rdma(src, dst, ssem, rsem, dev):
            return pltpu.make_async_remote_copy(
                src_ref=src, dst_ref=dst, send_sem=ssem, recv_sem=rsem,
                device_id=(dev,), device_id_type=pl.DeviceIdType.MESH,
            )

        for h in range(NP - 1):
            cw_s = lax.rem(qi - h + NP, NP) * MC
            cw_r = lax.rem(qi - h + 3, NP) * MC
            ccw_s = lax.rem(qi + h + 2, NP) * MC
            ccw_r = lax.rem(qi + h + 3, NP) * MC
            sb_cw[...] = p_ref[pl.ds(cw_s, MC), pl.ds(0, HC)].astype(bf16)
            sb_ccw[...] = p_ref[pl.ds(ccw_s, MC), pl.ds(HC, HC)].astype(bf16)
            cw = rdma(sb_cw, rb_cw.at[h], prs_send_cw.at[h],
                      prs_recv_cw.at[h], right)
            ccw = rdma(sb_ccw, rb_ccw.at[h], prs_send_ccw.at[h],
                       prs_recv_ccw.at[h], left)
            cw.start()
            ccw.start()
            cw.wait()
            ccw.wait()
            p_ref[pl.ds(cw_r, MC), pl.ds(0, HC)] += rb_cw[h].astype(f32)
            p_ref[pl.ds(ccw_r, MC), pl.ds(HC, HC)] += rb_ccw[h].astype(f32)

        base = lax.rem(qi + 1, NP) * MC

        keep_a = base + 128 * b0
        send_a = base + 128 * (1 - b0)
        zsa[...] = p_ref[pl.ds(send_a, 128), :].astype(bf16)
        ex = rdma(zsa, zra, z_send.at[0], z_recv.at[0], dz1)
        ex.start()
        ex.wait()
        p_ref[pl.ds(keep_a, 128), :] += zra[...].astype(f32)

        keep_b = keep_a + 64 * b1
        send_b = keep_a + 64 * (1 - b1)
        zsb[...] = p_ref[pl.ds(send_b, 64), :].astype(bf16)
        ex = rdma(zsb, zrb, z_send.at[1], z_recv.at[1], dz2)
        ex.start()
        ex.wait()
        p_ref[pl.ds(keep_b, 64), :] += zrb[...].astype(f32)

        r_ref[pl.ds(keep_b, 64), :] = p_ref[pl.ds(keep_b, 64), :].astype(bf16)

        ex = rdma(r_ref.at[pl.ds(keep_b, 64), :], r_ref.at[pl.ds(keep_b, 64), :],
                  z_send.at[2], z_recv.at[2], dz2)
        ex.start()
        ex.wait()

        ex = rdma(r_ref.at[pl.ds(keep_a, 128), :], r_ref.at[pl.ds(keep_a, 128), :],
                  z_send.at[3], z_recv.at[3], dz1)
        ex.start()
        ex.wait()

        for h in range(NP - 1):
            cw_rows = lax.rem(qi + 1 - h + NP, NP) * MC
            ccw_rows = lax.rem(qi + 1 + h, NP) * MC
            cw = rdma(r_ref.at[pl.ds(cw_rows, MC), pl.ds(0, HC)],
                      r_ref.at[pl.ds(cw_rows, MC), pl.ds(0, HC)],
                      pag_send_cw.at[h], pag_recv_cw.at[h], right)
            ccw = rdma(r_ref.at[pl.ds(ccw_rows, MC), pl.ds(HC, HC)],
                       r_ref.at[pl.ds(ccw_rows, MC), pl.ds(HC, HC)],
                       pag_send_ccw.at[h], pag_recv_ccw.at[h], left)
            cw.start()
            ccw.start()
            cw.wait()
            ccw.wait()

        out_ref[...] = r_ref[...].astype(f32)

        @functools.partial(pl.run_scoped, exit_sem=pltpu.SemaphoreType.REGULAR)
        def _(exit_sem):
            for nbr in (left, right, dz1, dz2):
                xsig(exit_sem, nbr)
            pl.semaphore_wait(exit_sem, 4)

    dma = pltpu.SemaphoreType.DMA
    return pl.pallas_call(
        body,
        out_shape=jax.ShapeDtypeStruct((M, N), f32),
        in_specs=[
            pl.BlockSpec(memory_space=pltpu.VMEM),
            pl.BlockSpec(memory_space=pltpu.VMEM),
        ],
        out_specs=pl.BlockSpec(memory_space=pltpu.VMEM),
        scratch_shapes=[
            pltpu.VMEM((M, N), f32),
            pltpu.VMEM((M, N), bf16),
            pltpu.VMEM((MC, HC), bf16),
            pltpu.VMEM((MC, HC), bf16),
            pltpu.VMEM((NP - 1, MC, HC), bf16),
            pltpu.VMEM((NP - 1, MC, HC), bf16),
            pltpu.VMEM((128, N), bf16),
            pltpu.VMEM((128, N), bf16),
            pltpu.VMEM((64, N), bf16),
            pltpu.VMEM((64, N), bf16),
            dma((NP - 1,)), dma((NP - 1,)),
            dma((NP - 1,)), dma((NP - 1,)),
            dma((4,)), dma((4,)),
            dma((NP - 1,)), dma((NP - 1,)),
            dma((NP - 1,)), dma((NP - 1,)),
        ],
        compiler_params=pltpu.CompilerParams(collective_id=0),
    )(A, B)


# device time: 57506 ns/iter; 2.6180x vs baseline; 1.0031x over previous
import functools

import jax
import jax.numpy as jnp
from jax import lax
from jax.experimental import pallas as pl
from jax.experimental.pallas import tpu as pltpu

N_DEV = 16
NP = 4
MC = 256
HC = 512


def kernel(A, B):
    M, _ = A.shape
    _, N = B.shape

    f32 = jnp.float32
    bf16 = jnp.bfloat16

    def body(
        a_ref, b_ref, out_ref,
        p_ref, r_ref, sb_cw, sb_ccw, rb_cw, rb_ccw,
        zsa, zra, zsb, zrb,
        prs_send_cw, prs_recv_cw, prs_send_ccw, prs_recv_ccw,
        z_send, z_recv,
        pag_send_cw, pag_recv_cw, pag_send_ccw, pag_recv_ccw,
    ):
        my = lax.axis_index("i")
        qi = lax.rem(my, NP)
        zi = lax.div(my, NP)
        b0 = lax.rem(zi, 2)
        b1 = lax.div(zi, 2)

        right = zi * NP + lax.rem(qi + 1, NP)
        left = zi * NP + lax.rem(qi + 3, NP)
        dz1 = (zi + 1 - 2 * b0) * NP + qi
        dz2 = (zi + 2 - 4 * b1) * NP + qi

        def xsig(sem, dev):
            pl.semaphore_signal(
                sem, inc=1, device_id=(dev,),
                device_id_type=pl.DeviceIdType.MESH,
            )

        barrier_sem = pltpu.get_barrier_semaphore()
        for nbr in (left, right, dz1, dz2):
            xsig(barrier_sem, nbr)
        pl.semaphore_wait(barrier_sem, 4)

        def rdma(src, dst, ssem, rsem, dev):
            return pltpu.make_async_remote_copy(
                src_ref=src, dst_ref=dst, send_sem=ssem, recv_sem=rsem,
                device_id=(dev,), device_id_type=pl.DeviceIdType.MESH,
            )

        a_bf = a_ref[...].astype(bf16)

        p_ref[:, pl.ds(0, HC)] = jnp.dot(
            a_bf, b_ref[:, pl.ds(0, HC)].astype(bf16),
            preferred_element_type=f32,
        )
        cw_s0 = lax.rem(qi, NP) * MC
        sb_cw[...] = p_ref[pl.ds(cw_s0, MC), pl.ds(0, HC)].astype(bf16)
        cw = rdma(sb_cw, rb_cw.at[0], prs_send_cw.at[0], prs_recv_cw.at[0],
                  right)
        cw.start()

        p_ref[:, pl.ds(HC, HC)] = jnp.dot(
            a_bf, b_ref[:, pl.ds(HC, HC)].astype(bf16),
            preferred_element_type=f32,
        )
        ccw_s0 = lax.rem(qi + 2, NP) * MC
        sb_ccw[...] = p_ref[pl.ds(ccw_s0, MC), pl.ds(HC, HC)].astype(bf16)
        ccw = rdma(sb_ccw, rb_ccw.at[0], prs_send_ccw.at[0],
                   prs_recv_ccw.at[0], left)
        ccw.start()

        base = lax.rem(qi + 1, NP) * MC

        for h in range(NP - 1):
            cw_r = lax.rem(qi - h + 3, NP) * MC
            ccw_r = lax.rem(qi + h + 3, NP) * MC
            cw.wait()
            if h < NP - 2:
                sb_cw[...] = (
                    p_ref[pl.ds(cw_r, MC), pl.ds(0, HC)]
                    + rb_cw[h].astype(f32)
                ).astype(bf16)
                cw = rdma(sb_cw, rb_cw.at[h + 1], prs_send_cw.at[h + 1],
                          prs_recv_cw.at[h + 1], right)
                cw.start()
            else:
                p_ref[pl.ds(base, MC), pl.ds(0, HC)] += rb_cw[h].astype(f32)
            ccw.wait()
            if h < NP - 2:
                sb_ccw[...] = (
                    p_ref[pl.ds(ccw_r, MC), pl.ds(HC, HC)]
                    + rb_ccw[h].astype(f32)
                ).astype(bf16)
                ccw = rdma(sb_ccw, rb_ccw.at[h + 1], prs_send_ccw.at[h + 1],
                           prs_recv_ccw.at[h + 1], left)
                ccw.start()
            else:
                p_ref[pl.ds(base, MC), pl.ds(HC, HC)] += rb_ccw[h].astype(f32)

        keep_a = base + 128 * b0
        send_a = base + 128 * (1 - b0)
        zsa[...] = p_ref[pl.ds(send_a, 128), :].astype(bf16)
        ex = rdma(zsa, zra, z_send.at[0], z_recv.at[0], dz1)
        ex.start()
        ex.wait()

        keep_b = keep_a + 64 * b1
        send_b = keep_a + 64 * (1 - b1)
        zsb[...] = (
            p_ref[pl.ds(send_b, 64), :]
            + zra[pl.ds(64 * (1 - b1), 64), :].astype(f32)
        ).astype(bf16)
        ex = rdma(zsb, zrb, z_send.at[1], z_recv.at[1], dz2)
        ex.start()
        ex.wait()

        red = (
            p_ref[pl.ds(keep_b, 64), :]
            + zra[pl.ds(64 * b1, 64), :].astype(f32)
            + zrb[...].astype(f32)
        )
        r_ref[pl.ds(keep_b, 64), :] = red.astype(bf16)
        out_ref[pl.ds(keep_b, 64), :] = red

        ex = rdma(r_ref.at[pl.ds(keep_b, 64), :], r_ref.at[pl.ds(keep_b, 64), :],
                  z_send.at[2], z_recv.at[2], dz2)
        ex.start()
        ex.wait()

        ex = rdma(r_ref.at[pl.ds(keep_a, 128), :], r_ref.at[pl.ds(keep_a, 128), :],
                  z_send.at[3], z_recv.at[3], dz1)
        ex.start()
        ex.wait()

        for h in range(NP - 1):
            cw_rows = lax.rem(qi + 1 - h + NP, NP) * MC
            ccw_rows = lax.rem(qi + 1 + h, NP) * MC
            cw = rdma(r_ref.at[pl.ds(cw_rows, MC), pl.ds(0, HC)],
                      r_ref.at[pl.ds(cw_rows, MC), pl.ds(0, HC)],
                      pag_send_cw.at[h], pag_recv_cw.at[h], right)
            ccw = rdma(r_ref.at[pl.ds(ccw_rows, MC), pl.ds(HC, HC)],
                       r_ref.at[pl.ds(ccw_rows, MC), pl.ds(HC, HC)],
                       pag_send_ccw.at[h], pag_recv_ccw.at[h], left)
            cw.start()
            ccw.start()
            if h == 0:
                out_ref[pl.ds(keep_a + 64 * (1 - b1), 64), :] = (
                    r_ref[pl.ds(keep_a + 64 * (1 - b1), 64), :].astype(f32)
                )
                other_half = base + 128 * (1 - b0)
                out_ref[pl.ds(other_half, 128), :] = (
                    r_ref[pl.ds(other_half, 128), :].astype(f32)
                )
            else:
                pcw = lax.rem(qi - h + 1 + NP, NP) * MC
                pccw = lax.rem(qi + h + 1, NP) * MC
                out_ref[pl.ds(pcw, MC), pl.ds(0, HC)] = (
                    r_ref[pl.ds(pcw, MC), pl.ds(0, HC)].astype(f32)
                )
                out_ref[pl.ds(pccw, MC), pl.ds(HC, HC)] = (
                    r_ref[pl.ds(pccw, MC), pl.ds(HC, HC)].astype(f32)
                )
            cw.wait()
            ccw.wait()

        pcw = lax.rem(qi + 2, NP) * MC
        pccw = qi * MC
        out_ref[pl.ds(pcw, MC), pl.ds(0, HC)] = (
            r_ref[pl.ds(pcw, MC), pl.ds(0, HC)].astype(f32)
        )
        out_ref[pl.ds(pccw, MC), pl.ds(HC, HC)] = (
            r_ref[pl.ds(pccw, MC), pl.ds(HC, HC)].astype(f32)
        )

        @functools.partial(pl.run_scoped, exit_sem=pltpu.SemaphoreType.REGULAR)
        def _(exit_sem):
            for nbr in (left, right, dz1, dz2):
                xsig(exit_sem, nbr)
            pl.semaphore_wait(exit_sem, 4)

    dma = pltpu.SemaphoreType.DMA
    return pl.pallas_call(
        body,
        out_shape=jax.ShapeDtypeStruct((M, N), f32),
        in_specs=[
            pl.BlockSpec(memory_space=pltpu.VMEM),
            pl.BlockSpec(memory_space=pltpu.VMEM),
        ],
        out_specs=pl.BlockSpec(memory_space=pltpu.VMEM),
        scratch_shapes=[
            pltpu.VMEM((M, N), f32),
            pltpu.VMEM((M, N), bf16),
            pltpu.VMEM((MC, HC), bf16),
            pltpu.VMEM((MC, HC), bf16),
            pltpu.VMEM((NP - 1, MC, HC), bf16),
            pltpu.VMEM((NP - 1, MC, HC), bf16),
            pltpu.VMEM((128, N), bf16),
            pltpu.VMEM((128, N), bf16),
            pltpu.VMEM((64, N), bf16),
            pltpu.VMEM((64, N), bf16),
            dma((NP - 1,)), dma((NP - 1,)),
            dma((NP - 1,)), dma((NP - 1,)),
            dma((4,)), dma((4,)),
            dma((NP - 1,)), dma((NP - 1,)),
            dma((NP - 1,)), dma((NP - 1,)),
        ],
        compiler_params=pltpu.CompilerParams(collective_id=0),
    )(A, B)


# device time: 53488 ns/iter; 2.8147x vs baseline; 1.0751x over previous
import functools

import jax
import jax.numpy as jnp
from jax import lax
from jax.experimental import pallas as pl
from jax.experimental.pallas import tpu as pltpu

N_DEV = 16
NP = 4
MC = 256
HF = 128
HC = 512


def kernel(A, B):
    M, _ = A.shape
    _, N = B.shape

    f32 = jnp.float32
    bf16 = jnp.bfloat16

    def body(
        a_ref, b_ref, out_ref,
        p_ref, r_ref, sb_cw, sb_ccw, rb_cw, rb_ccw,
        zsa_l, zsa_r, zra_l, zra_r, zsb, zrb,
        snd_cw1, snd_cw2, snd_ccw1, snd_ccw2,
        rcv_cw1, rcv_cw2, rcv_ccw1, rcv_ccw2,
        z_snd, z_rcv,
        asnd_cw1, asnd_cw2, asnd_ccw1, asnd_ccw2,
        arcv_cw1, arcv_cw2, arcv_ccw1, arcv_ccw2,
    ):
        my = lax.axis_index("i")
        qi = lax.rem(my, NP)
        zi = lax.div(my, NP)
        b0 = lax.rem(zi, 2)
        b1 = lax.div(zi, 2)

        right = zi * NP + lax.rem(qi + 1, NP)
        left = zi * NP + lax.rem(qi + 3, NP)
        dz1 = (zi + 1 - 2 * b0) * NP + qi
        dz2 = (zi + 2 - 4 * b1) * NP + qi

        def xsig(sem, dev):
            pl.semaphore_signal(
                sem, inc=1, device_id=(dev,),
                device_id_type=pl.DeviceIdType.MESH,
            )

        barrier_sem = pltpu.get_barrier_semaphore()
        for nbr in (left, right, dz1, dz2):
            xsig(barrier_sem, nbr)
        pl.semaphore_wait(barrier_sem, 4)

        def rdma(src, dst, ssem, rsem, dev):
            return pltpu.make_async_remote_copy(
                src_ref=src, dst_ref=dst, send_sem=ssem, recv_sem=rsem,
                device_id=(dev,), device_id_type=pl.DeviceIdType.MESH,
            )

        a_bf = a_ref[...].astype(bf16)

        p_ref[:, pl.ds(0, HC)] = jnp.dot(
            a_bf, b_ref[:, pl.ds(0, HC)].astype(bf16),
            preferred_element_type=f32,
        )
        cw_s0 = lax.rem(qi, NP) * MC
        sb_cw[...] = p_ref[pl.ds(cw_s0, MC), pl.ds(0, HC)].astype(bf16)
        cw1 = rdma(sb_cw.at[pl.ds(0, HF), :], rb_cw.at[0, pl.ds(0, HF), :],
                   snd_cw1, rcv_cw1.at[0], right)
        cw2 = rdma(sb_cw.at[pl.ds(HF, HF), :], rb_cw.at[0, pl.ds(HF, HF), :],
                   snd_cw2, rcv_cw2.at[0], right)
        cw1.start()
        cw2.start()

        p_ref[:, pl.ds(HC, HC)] = jnp.dot(
            a_bf, b_ref[:, pl.ds(HC, HC)].astype(bf16),
            preferred_element_type=f32,
        )
        ccw_s0 = lax.rem(qi + 2, NP) * MC
        sb_ccw[...] = p_ref[pl.ds(ccw_s0, MC), pl.ds(HC, HC)].astype(bf16)
        ccw1 = rdma(sb_ccw.at[pl.ds(0, HF), :], rb_ccw.at[0, pl.ds(0, HF), :],
                    snd_ccw1, rcv_ccw1.at[0], left)
        ccw2 = rdma(sb_ccw.at[pl.ds(HF, HF), :], rb_ccw.at[0, pl.ds(HF, HF), :],
                    snd_ccw2, rcv_ccw2.at[0], left)
        ccw1.start()
        ccw2.start()

        base = lax.rem(qi + 1, NP) * MC

        for h in range(NP - 2):
            cw_r = lax.rem(qi - h + 3, NP) * MC
            ccw_r = lax.rem(qi + h + 3, NP) * MC

            def fwd(desc, sb, rb, rows, c0, off, ssem, rsems, dev):
                desc.wait_recv()
                desc.wait_send()
                sb[pl.ds(off, HF), :] = (
                    p_ref[pl.ds(rows + off, HF), pl.ds(c0, HC)]
                    + rb[h, pl.ds(off, HF), :].astype(f32)
                ).astype(bf16)
                nxt = rdma(sb.at[pl.ds(off, HF), :],
                           rb.at[h + 1, pl.ds(off, HF), :],
                           ssem, rsems.at[h + 1], dev)
                nxt.start()
                return nxt

            cw1 = fwd(cw1, sb_cw, rb_cw, cw_r, 0, 0, snd_cw1, rcv_cw1, right)
            ccw1 = fwd(ccw1, sb_ccw, rb_ccw, ccw_r, HC, 0, snd_ccw1,
                       rcv_ccw1, left)
            cw2 = fwd(cw2, sb_cw, rb_cw, cw_r, 0, HF, snd_cw2, rcv_cw2, right)
            ccw2 = fwd(ccw2, sb_ccw, rb_ccw, ccw_r, HC, HF, snd_ccw2,
                       rcv_ccw2, left)

        keep_a = base + 128 * b0
        send_a = base + 128 * (1 - b0)
        H = NP - 2

        cw1.wait_recv()
        cw2.wait_recv()
        zsa_l[...] = (
            p_ref[pl.ds(send_a, HF), pl.ds(0, HC)]
            + rb_cw[H, pl.ds(128 * (1 - b0), HF), :].astype(f32)
        ).astype(bf16)
        za_l = rdma(zsa_l, zra_l, z_snd.at[0], z_rcv.at[0], dz1)
        za_l.start()
        p_ref[pl.ds(keep_a, HF), pl.ds(0, HC)] += (
            rb_cw[H, pl.ds(128 * b0, HF), :].astype(f32)
        )

        ccw1.wait_recv()
        ccw2.wait_recv()
        zsa_r[...] = (
            p_ref[pl.ds(send_a, HF), pl.ds(HC, HC)]
            + rb_ccw[H, pl.ds(128 * (1 - b0), HF), :].astype(f32)
        ).astype(bf16)
        za_r = rdma(zsa_r, zra_r, z_snd.at[1], z_rcv.at[1], dz1)
        za_r.start()
        p_ref[pl.ds(keep_a, HF), pl.ds(HC, HC)] += (
            rb_ccw[H, pl.ds(128 * b0, HF), :].astype(f32)
        )

        keep_b = keep_a + 64 * b1
        send_b = keep_a + 64 * (1 - b1)
        za_l.wait_recv()
        za_r.wait_recv()
        zsb[:, pl.ds(0, HC)] = (
            p_ref[pl.ds(send_b, 64), pl.ds(0, HC)]
            + zra_l[pl.ds(64 * (1 - b1), 64), :].astype(f32)
        ).astype(bf16)
        zsb[:, pl.ds(HC, HC)] = (
            p_ref[pl.ds(send_b, 64), pl.ds(HC, HC)]
            + zra_r[pl.ds(64 * (1 - b1), 64), :].astype(f32)
        ).astype(bf16)
        zb = rdma(zsb, zrb, z_snd.at[2], z_rcv.at[2], dz2)
        zb.start()
        p_ref[pl.ds(keep_b, 64), pl.ds(0, HC)] += (
            zra_l[pl.ds(64 * b1, 64), :].astype(f32)
        )
        p_ref[pl.ds(keep_b, 64), pl.ds(HC, HC)] += (
            zra_r[pl.ds(64 * b1, 64), :].astype(f32)
        )
        zb.wait_recv()
        red = p_ref[pl.ds(keep_b, 64), :] + zrb[...].astype(f32)
        r_ref[pl.ds(keep_b, 64), :] = red.astype(bf16)
        out_ref[pl.ds(keep_b, 64), :] = red

        zc = rdma(r_ref.at[pl.ds(keep_b, 64), :], r_ref.at[pl.ds(keep_b, 64), :],
                  z_snd.at[3], z_rcv.at[3], dz2)
        zc.start()
        zc.wait_recv()

        zd = rdma(r_ref.at[pl.ds(keep_a, HF), :], r_ref.at[pl.ds(keep_a, HF), :],
                  z_snd.at[4], z_rcv.at[4], dz1)
        zd.start()
        zd.wait_recv()

        def ag(rows, off, c0, ssem, rsem, dev):
            d = rdma(r_ref.at[pl.ds(rows + off, HF), pl.ds(c0, HC)],
                     r_ref.at[pl.ds(rows + off, HF), pl.ds(c0, HC)],
                     ssem, rsem, dev)
            d.start()
            return d

        g_cw = base
        g_ccw = base
        acw1 = ag(g_cw, 0, 0, asnd_cw1, arcv_cw1.at[0], right)
        accw1 = ag(g_ccw, 0, HC, asnd_ccw1, arcv_ccw1.at[0], left)
        acw2 = ag(g_cw, HF, 0, asnd_cw2, arcv_cw2.at[0], right)
        accw2 = ag(g_ccw, HF, HC, asnd_ccw2, arcv_ccw2.at[0], left)

        for h in range(NP - 1):
            if h == 0:
                out_ref[pl.ds(keep_a + 64 * (1 - b1), 64), :] = (
                    r_ref[pl.ds(keep_a + 64 * (1 - b1), 64), :].astype(f32)
                )
                oh = base + 128 * (1 - b0)
                out_ref[pl.ds(oh, HF), :] = r_ref[pl.ds(oh, HF), :].astype(f32)
            else:
                pcw = lax.rem(qi - h + 1 + NP, NP) * MC
                pccw = lax.rem(qi + h + 1, NP) * MC
                out_ref[pl.ds(pcw, MC), pl.ds(0, HC)] = (
                    r_ref[pl.ds(pcw, MC), pl.ds(0, HC)].astype(f32)
                )
                out_ref[pl.ds(pccw, MC), pl.ds(HC, HC)] = (
                    r_ref[pl.ds(pccw, MC), pl.ds(HC, HC)].astype(f32)
                )

            n_cw = lax.rem(qi - h + NP, NP) * MC
            n_ccw = lax.rem(qi + h + 2, NP) * MC
            acw1.wait_recv()
            accw1.wait_recv()
            acw2.wait_recv()
            accw2.wait_recv()
            if h < NP - 2:
                acw1.wait_send()
                acw1 = ag(n_cw, 0, 0, asnd_cw1, arcv_cw1.at[h + 1], right)
                accw1.wait_send()
                accw1 = ag(n_ccw, 0, HC, asnd_ccw1, arcv_ccw1.at[h + 1], left)
                acw2.wait_send()
                acw2 = ag(n_cw, HF, 0, asnd_cw2, arcv_cw2.at[h + 1], right)
                accw2.wait_send()
                accw2 = ag(n_ccw, HF, HC, asnd_ccw2, arcv_ccw2.at[h + 1], left)

        pcw = lax.rem(qi + 2, NP) * MC
        pccw = qi * MC
        out_ref[pl.ds(pcw, MC), pl.ds(0, HC)] = (
            r_ref[pl.ds(pcw, MC), pl.ds(0, HC)].astype(f32)
        )
        out_ref[pl.ds(pccw, MC), pl.ds(HC, HC)] = (
            r_ref[pl.ds(pccw, MC), pl.ds(HC, HC)].astype(f32)
        )

        for d in (cw1, cw2, ccw1, ccw2, za_l, za_r, zb, zc, zd,
                  acw1, accw1, acw2, accw2):
            d.wait_send()

        @functools.partial(pl.run_scoped, exit_sem=pltpu.SemaphoreType.REGULAR)
        def _(exit_sem):
            for nbr in (left, right, dz1, dz2):
                xsig(exit_sem, nbr)
            pl.semaphore_wait(exit_sem, 4)

    dma = pltpu.SemaphoreType.DMA
    return pl.pallas_call(
        body,
        out_shape=jax.ShapeDtypeStruct((M, N), f32),
        in_specs=[
            pl.BlockSpec(memory_space=pltpu.VMEM),
            pl.BlockSpec(memory_space=pltpu.VMEM),
        ],
        out_specs=pl.BlockSpec(memory_space=pltpu.VMEM),
        scratch_shapes=[
            pltpu.VMEM((M, N), f32),
            pltpu.VMEM((M, N), bf16),
            pltpu.VMEM((MC, HC), bf16),
            pltpu.VMEM((MC, HC), bf16),
            pltpu.VMEM((NP - 1, MC, HC), bf16),
            pltpu.VMEM((NP - 1, MC, HC), bf16),
            pltpu.VMEM((HF, HC), bf16),
            pltpu.VMEM((HF, HC), bf16),
            pltpu.VMEM((HF, HC), bf16),
            pltpu.VMEM((HF, HC), bf16),
            pltpu.VMEM((64, N), bf16),
            pltpu.VMEM((64, N), bf16),
            dma, dma, dma, dma,
            dma((NP - 1,)), dma((NP - 1,)),
            dma((NP - 1,)), dma((NP - 1,)),
            dma((5,)), dma((5,)),
            dma, dma, dma, dma,
            dma((NP - 1,)), dma((NP - 1,)),
            dma((NP - 1,)), dma((NP - 1,)),
        ],
        compiler_params=pltpu.CompilerParams(collective_id=0),
    )(A, B)


# device time: 48928 ns/iter; 3.0770x vs baseline; 1.0932x over previous
import functools

import jax
import jax.numpy as jnp
from jax import lax
from jax.experimental import pallas as pl
from jax.experimental.pallas import tpu as pltpu

N_DEV = 16
NP = 4
MC = 256
HF = 128
HC = 512


def kernel(A, B):
    M, _ = A.shape
    _, N = B.shape

    f32 = jnp.float32
    bf16 = jnp.bfloat16

    def body(
        a_ref, b_ref, out_ref,
        p_ref, sb_cw, sb_ccw, rb_cw, rb_ccw,
        zsa_l, zsa_r, zra_l, zra_r, zsb, zrb,
        snd_cw1, snd_cw2, snd_ccw1, snd_ccw2,
        rcv_cw1, rcv_cw2, rcv_ccw1, rcv_ccw2,
        z_snd, z_rcv,
        asnd_cw1, asnd_cw2, asnd_ccw1, asnd_ccw2,
        arcv_cw1, arcv_cw2, arcv_ccw1, arcv_ccw2,
    ):
        my = lax.axis_index("i")
        qi = lax.rem(my, NP)
        zi = lax.div(my, NP)
        b0 = lax.rem(zi, 2)
        b1 = lax.div(zi, 2)

        right = zi * NP + lax.rem(qi + 1, NP)
        left = zi * NP + lax.rem(qi + 3, NP)
        dz1 = (zi + 1 - 2 * b0) * NP + qi
        dz2 = (zi + 2 - 4 * b1) * NP + qi

        def xsig(sem, dev):
            pl.semaphore_signal(
                sem, inc=1, device_id=(dev,),
                device_id_type=pl.DeviceIdType.MESH,
            )

        barrier_sem = pltpu.get_barrier_semaphore()
        for nbr in (left, right, dz1, dz2):
            xsig(barrier_sem, nbr)
        pl.semaphore_wait(barrier_sem, 4)

        def rdma(src, dst, ssem, rsem, dev):
            return pltpu.make_async_remote_copy(
                src_ref=src, dst_ref=dst, send_sem=ssem, recv_sem=rsem,
                device_id=(dev,), device_id_type=pl.DeviceIdType.MESH,
            )

        a_bf = a_ref[...].astype(bf16)

        p_ref[:, pl.ds(0, HC)] = jnp.dot(
            a_bf, b_ref[:, pl.ds(0, HC)].astype(bf16),
            preferred_element_type=f32,
        )
        cw_s0 = lax.rem(qi, NP) * MC
        sb_cw[...] = p_ref[pl.ds(cw_s0, MC), pl.ds(0, HC)].astype(bf16)
        cw1 = rdma(sb_cw.at[pl.ds(0, HF), :], rb_cw.at[0, pl.ds(0, HF), :],
                   snd_cw1, rcv_cw1.at[0], right)
        cw2 = rdma(sb_cw.at[pl.ds(HF, HF), :], rb_cw.at[0, pl.ds(HF, HF), :],
                   snd_cw2, rcv_cw2.at[0], right)
        cw1.start()
        cw2.start()

        p_ref[:, pl.ds(HC, HC)] = jnp.dot(
            a_bf, b_ref[:, pl.ds(HC, HC)].astype(bf16),
            preferred_element_type=f32,
        )
        ccw_s0 = lax.rem(qi + 2, NP) * MC
        sb_ccw[...] = p_ref[pl.ds(ccw_s0, MC), pl.ds(HC, HC)].astype(bf16)
        ccw1 = rdma(sb_ccw.at[pl.ds(0, HF), :], rb_ccw.at[0, pl.ds(0, HF), :],
                    snd_ccw1, rcv_ccw1.at[0], left)
        ccw2 = rdma(sb_ccw.at[pl.ds(HF, HF), :], rb_ccw.at[0, pl.ds(HF, HF), :],
                    snd_ccw2, rcv_ccw2.at[0], left)
        ccw1.start()
        ccw2.start()

        base = lax.rem(qi + 1, NP) * MC

        for h in range(NP - 2):
            cw_r = lax.rem(qi - h + 3, NP) * MC
            ccw_r = lax.rem(qi + h + 3, NP) * MC

            def fwd(desc, sb, rb, rows, c0, off, ssem, rsems, dev):
                desc.wait_recv()
                desc.wait_send()
                sb[pl.ds(off, HF), :] = (
                    p_ref[pl.ds(rows + off, HF), pl.ds(c0, HC)]
                    + rb[h, pl.ds(off, HF), :].astype(f32)
                ).astype(bf16)
                nxt = rdma(sb.at[pl.ds(off, HF), :],
                           rb.at[h + 1, pl.ds(off, HF), :],
                           ssem, rsems.at[h + 1], dev)
                nxt.start()
                return nxt

            cw1 = fwd(cw1, sb_cw, rb_cw, cw_r, 0, 0, snd_cw1, rcv_cw1, right)
            ccw1 = fwd(ccw1, sb_ccw, rb_ccw, ccw_r, HC, 0, snd_ccw1,
                       rcv_ccw1, left)
            cw2 = fwd(cw2, sb_cw, rb_cw, cw_r, 0, HF, snd_cw2, rcv_cw2, right)
            ccw2 = fwd(ccw2, sb_ccw, rb_ccw, ccw_r, HC, HF, snd_ccw2,
                       rcv_ccw2, left)

        keep_a = base + 128 * b0
        send_a = base + 128 * (1 - b0)
        H = NP - 2

        cw1.wait_recv()
        cw2.wait_recv()
        zsa_l[...] = (
            p_ref[pl.ds(send_a, HF), pl.ds(0, HC)]
            + rb_cw[H, pl.ds(128 * (1 - b0), HF), :].astype(f32)
        ).astype(bf16)
        za_l = rdma(zsa_l, zra_l, z_snd.at[0], z_rcv.at[0], dz1)
        za_l.start()
        p_ref[pl.ds(keep_a, HF), pl.ds(0, HC)] += (
            rb_cw[H, pl.ds(128 * b0, HF), :].astype(f32)
        )

        ccw1.wait_recv()
        ccw2.wait_recv()
        zsa_r[...] = (
            p_ref[pl.ds(send_a, HF), pl.ds(HC, HC)]
            + rb_ccw[H, pl.ds(128 * (1 - b0), HF), :].astype(f32)
        ).astype(bf16)
        za_r = rdma(zsa_r, zra_r, z_snd.at[1], z_rcv.at[1], dz1)
        za_r.start()
        p_ref[pl.ds(keep_a, HF), pl.ds(HC, HC)] += (
            rb_ccw[H, pl.ds(128 * b0, HF), :].astype(f32)
        )

        keep_b = keep_a + 64 * b1
        send_b = keep_a + 64 * (1 - b1)
        za_l.wait_recv()
        za_r.wait_recv()
        zsb[:, pl.ds(0, HC)] = (
            p_ref[pl.ds(send_b, 64), pl.ds(0, HC)]
            + zra_l[pl.ds(64 * (1 - b1), 64), :].astype(f32)
        ).astype(bf16)
        zsb[:, pl.ds(HC, HC)] = (
            p_ref[pl.ds(send_b, 64), pl.ds(HC, HC)]
            + zra_r[pl.ds(64 * (1 - b1), 64), :].astype(f32)
        ).astype(bf16)
        zb = rdma(zsb, zrb, z_snd.at[2], z_rcv.at[2], dz2)
        zb.start()
        p_ref[pl.ds(keep_b, 64), pl.ds(0, HC)] += (
            zra_l[pl.ds(64 * b1, 64), :].astype(f32)
        )
        p_ref[pl.ds(keep_b, 64), pl.ds(HC, HC)] += (
            zra_r[pl.ds(64 * b1, 64), :].astype(f32)
        )
        zb.wait_recv()
        out_ref[pl.ds(keep_b, 64), :] = (
            p_ref[pl.ds(keep_b, 64), :] + zrb[...].astype(f32)
        ).astype(bf16)

        zc = rdma(out_ref.at[pl.ds(keep_b, 64), :],
                  out_ref.at[pl.ds(keep_b, 64), :],
                  z_snd.at[3], z_rcv.at[3], dz2)
        zc.start()
        zc.wait_recv()

        def ag(rows, off, c0, ssem, rsem, dev):
            d = rdma(out_ref.at[pl.ds(rows + off, HF), pl.ds(c0, HC)],
                     out_ref.at[pl.ds(rows + off, HF), pl.ds(c0, HC)],
                     ssem, rsem, dev)
            d.start()
            return d

        off1 = 128 * b0
        off2 = 128 * (1 - b0)
        acw1 = ag(base, off1, 0, asnd_cw1, arcv_cw1.at[0], right)
        accw1 = ag(base, off1, HC, asnd_ccw1, arcv_ccw1.at[0], left)

        zd = rdma(out_ref.at[pl.ds(keep_a, HF), :],
                  out_ref.at[pl.ds(keep_a, HF), :],
                  z_snd.at[4], z_rcv.at[4], dz1)
        zd.start()
        zd.wait_recv()

        acw2 = ag(base, off2, 0, asnd_cw2, arcv_cw2.at[0], right)
        accw2 = ag(base, off2, HC, asnd_ccw2, arcv_ccw2.at[0], left)

        for h in range(NP - 1):
            n_cw = lax.rem(qi - h + NP, NP) * MC
            n_ccw = lax.rem(qi + h + 2, NP) * MC
            acw1.wait_recv()
            accw1.wait_recv()
            if h < NP - 2:
                acw1.wait_send()
                acw1 = ag(n_cw, off1, 0, asnd_cw1, arcv_cw1.at[h + 1], right)
                accw1.wait_send()
                accw1 = ag(n_ccw, off1, HC, asnd_ccw1, arcv_ccw1.at[h + 1],
                           left)
            acw2.wait_recv()
            accw2.wait_recv()
            if h < NP - 2:
                acw2.wait_send()
                acw2 = ag(n_cw, off2, 0, asnd_cw2, arcv_cw2.at[h + 1], right)
                accw2.wait_send()
                accw2 = ag(n_ccw, off2, HC, asnd_ccw2, arcv_ccw2.at[h + 1],
                           left)

        for d in (cw1, cw2, ccw1, ccw2, za_l, za_r, zb, zc, zd,
                  acw1, accw1, acw2, accw2):
            d.wait_send()

        @functools.partial(pl.run_scoped, exit_sem=pltpu.SemaphoreType.REGULAR)
        def _(exit_sem):
            for nbr in (left, right, dz1, dz2):
                xsig(exit_sem, nbr)
            pl.semaphore_wait(exit_sem, 4)

    dma = pltpu.SemaphoreType.DMA
    return pl.pallas_call(
        body,
        out_shape=jax.ShapeDtypeStruct((M, N), bf16),
        in_specs=[
            pl.BlockSpec(memory_space=pltpu.VMEM),
            pl.BlockSpec(memory_space=pltpu.VMEM),
        ],
        out_specs=pl.BlockSpec(memory_space=pltpu.VMEM),
        scratch_shapes=[
            pltpu.VMEM((M, N), f32),
            pltpu.VMEM((MC, HC), bf16),
            pltpu.VMEM((MC, HC), bf16),
            pltpu.VMEM((NP - 1, MC, HC), bf16),
            pltpu.VMEM((NP - 1, MC, HC), bf16),
            pltpu.VMEM((HF, HC), bf16),
            pltpu.VMEM((HF, HC), bf16),
            pltpu.VMEM((HF, HC), bf16),
            pltpu.VMEM((HF, HC), bf16),
            pltpu.VMEM((64, N), bf16),
            pltpu.VMEM((64, N), bf16),
            dma, dma, dma, dma,
            dma((NP - 1,)), dma((NP - 1,)),
            dma((NP - 1,)), dma((NP - 1,)),
            dma((5,)), dma((5,)),
            dma, dma, dma, dma,
            dma((NP - 1,)), dma((NP - 1,)),
            dma((NP - 1,)), dma((NP - 1,)),
        ],
        compiler_params=pltpu.CompilerParams(collective_id=0),
    )(A, B)
